# baseline (speedup 1.0000x reference)
import functools

import jax
import jax.numpy as jnp
from jax import lax
from jax.experimental import pallas as pl
from jax.experimental.pallas import tpu as pltpu
from jax.experimental.pallas import tpu_sc as plsc

EMB = 64
BLK = 128
TROW = 128


@functools.cache
def _make_gather(n_b: int, n_t: int):
    info = plsc.get_sparse_core_info()
    nc, ns = info.num_cores, info.num_subcores
    nw = nc * ns
    assert n_b == nw * BLK and n_t % 2 == 0
    mesh = plsc.VectorSubcoreMesh(core_axis_name="c", subcore_axis_name="s")

    @functools.partial(
        pl.kernel,
        out_type=jax.ShapeDtypeStruct((n_t, EMB, n_b), jnp.float32),
        mesh=mesh,
        scratch_types=[
            pltpu.VMEM((n_t, BLK), jnp.int32),
            pltpu.VMEM((2, BLK), jnp.int32),
            pltpu.VMEM((2, BLK), jnp.int32),
            pltpu.VMEM((2, BLK, TROW), jnp.float32),
            pltpu.VMEM((2, EMB, BLK), jnp.float32),
            pltpu.SemaphoreType.DMA,
            pltpu.SemaphoreType.DMA,
            pltpu.SemaphoreType.DMA,
            pltpu.SemaphoreType.DMA,
        ],
        compiler_params=pltpu.CompilerParams(
            use_tc_tiling_on_sc=True, needs_layout_passes=False),
    )
    def gather_kernel(idxt_hbm, table_hbm, out_hbm, idx_v, rows_v, offs_v,
                      gbuf, sbuf, gsem0, gsem1, ssem0, ssem1):
        wid = lax.axis_index("s") * nc + lax.axis_index("c")
        b0 = wid * BLK
        pltpu.sync_copy(idxt_hbm.at[:, pl.ds(b0, BLK)], idx_v)

        gsems = (gsem0, gsem1)
        ssems = (ssem0, ssem1)
        iota = lax.iota(jnp.int32, 16)

        def prep(t, p):
            for g in range(BLK // 16):
                v = plsc.load_gather(idx_v, [jnp.full((16,), t, jnp.int32),
                                             g * 16 + iota])
                rp = rows_v.at[p]
                op_ = offs_v.at[p]
                rp[pl.ds(g * 16, 16)] = v >> 1
                op_[pl.ds(g * 16, 16)] = (v & 1) * EMB
            pltpu.async_copy(table_hbm.at[rows_v.at[p]], gbuf.at[p], gsems[p])

        def process(t, p):
            pltpu.make_async_copy(
                table_hbm.at[pl.ds(0, BLK)], gbuf.at[p], gsems[p]).wait()
            gp = gbuf.at[p]
            sp = sbuf.at[p]
            for g in range(BLK // 16):
                offg = offs_v[p, pl.ds(g * 16, 16)]
                rowg = g * 16 + iota
                for e in range(EMB):
                    val = plsc.load_gather(gp, [rowg, offg + e])
                    sp[e, pl.ds(g * 16, 16)] = val

            pltpu.async_copy(sp, out_hbm.at[t, :, pl.ds(b0, BLK)], ssems[p])

        prep(0, 0)

        @pl.loop(0, n_t, step=2)
        def _(t0):
            prep(t0 + 1, 1)

            @pl.when(t0 > 0)
            def _():
                pltpu.make_async_copy(
                    sbuf.at[0], out_hbm.at[0, :, pl.ds(0, BLK)], ssem0).wait()
            process(t0, 0)

            @pl.when(t0 + 2 < n_t)
            def _():
                prep(t0 + 2, 0)

            @pl.when(t0 > 0)
            def _():
                pltpu.make_async_copy(
                    sbuf.at[1], out_hbm.at[0, :, pl.ds(0, BLK)], ssem1).wait()
            process(t0 + 1, 1)

        pltpu.make_async_copy(
            sbuf.at[0], out_hbm.at[0, :, pl.ds(0, BLK)], ssem0).wait()
        pltpu.make_async_copy(
            sbuf.at[1], out_hbm.at[0, :, pl.ds(0, BLK)], ssem1).wait()

    return gather_kernel


def kernel(inputs, table):
    b, t = inputs.shape
    idxt = inputs.T.astype(jnp.int32)
    tablep = table.reshape(-1, TROW)
    outp = _make_gather(b, t)(idxt, tablep)
    return jnp.transpose(outp, (2, 0, 1))


# speedup vs baseline: 2.3381x; 2.3381x over previous
import functools

import jax
import jax.numpy as jnp
from jax import lax
from jax.experimental import pallas as pl
from jax.experimental.pallas import tpu as pltpu
from jax.experimental.pallas import tpu_sc as plsc

EMB = 64
BLK = 128
TROW = 128


@functools.cache
def _make_gather(n_b: int, n_t: int):
    info = plsc.get_sparse_core_info()
    nc, ns = info.num_cores, info.num_subcores
    nw = nc * ns
    assert n_b == nw * BLK and n_t % 2 == 0
    mesh = plsc.VectorSubcoreMesh(core_axis_name="c", subcore_axis_name="s")

    @functools.partial(
        pl.kernel,
        out_type=jax.ShapeDtypeStruct((n_t, EMB, n_b), jnp.float32),
        mesh=mesh,
        scratch_types=[
            pltpu.VMEM((n_t, BLK), jnp.int32),
            pltpu.VMEM((2, BLK), jnp.int32),
            pltpu.VMEM((2, BLK), jnp.int32),
            pltpu.VMEM((2, BLK, TROW), jnp.float32),
            pltpu.VMEM((2, EMB, BLK), jnp.float32),
            pltpu.SemaphoreType.DMA,
            pltpu.SemaphoreType.DMA,
            pltpu.SemaphoreType.DMA,
            pltpu.SemaphoreType.DMA,
        ],
        compiler_params=pltpu.CompilerParams(
            use_tc_tiling_on_sc=True, needs_layout_passes=False),
    )
    def gather_kernel(idxt_hbm, table_hbm, out_hbm, idx_v, rows_v, offs_v,
                      gbuf, sbuf, gsem0, gsem1, ssem0, ssem1):
        wid = lax.axis_index("s") * nc + lax.axis_index("c")
        b0 = wid * BLK
        pltpu.sync_copy(idxt_hbm.at[:, pl.ds(b0, BLK)], idx_v)

        gsems = (gsem0, gsem1)
        ssems = (ssem0, ssem1)
        iota = lax.iota(jnp.int32, 16)

        def prep(t, p):
            for g in range(BLK // 16):
                v = plsc.load_gather(idx_v, [jnp.full((16,), t, jnp.int32),
                                             g * 16 + iota])
                rp = rows_v.at[p]
                op_ = offs_v.at[p]
                rp[pl.ds(g * 16, 16)] = v >> 1
                op_[pl.ds(g * 16, 16)] = (v & 1) * EMB
            pltpu.async_copy(table_hbm.at[rows_v.at[p]], gbuf.at[p], gsems[p])

        def process(t, p):
            pltpu.make_async_copy(
                table_hbm.at[pl.ds(0, BLK)], gbuf.at[p], gsems[p]).wait()
            gp = gbuf.at[p]
            sp = sbuf.at[p]
            for g in range(BLK // 16):
                offg = offs_v[p, pl.ds(g * 16, 16)]
                rowg = g * 16 + iota

                @functools.partial(plsc.parallel_loop, 0, EMB, unroll=8)
                def _(e):
                    val = plsc.load_gather(gp, [rowg, offg + e])
                    sp[e, pl.ds(g * 16, 16)] = val

            pltpu.async_copy(sp, out_hbm.at[t, :, pl.ds(b0, BLK)], ssems[p])

        prep(0, 0)

        @pl.loop(0, n_t, step=2)
        def _(t0):
            prep(t0 + 1, 1)

            @pl.when(t0 > 0)
            def _():
                pltpu.make_async_copy(
                    sbuf.at[0], out_hbm.at[0, :, pl.ds(0, BLK)], ssem0).wait()
            process(t0, 0)

            @pl.when(t0 + 2 < n_t)
            def _():
                prep(t0 + 2, 0)

            @pl.when(t0 > 0)
            def _():
                pltpu.make_async_copy(
                    sbuf.at[1], out_hbm.at[0, :, pl.ds(0, BLK)], ssem1).wait()
            process(t0 + 1, 1)

        pltpu.make_async_copy(
            sbuf.at[0], out_hbm.at[0, :, pl.ds(0, BLK)], ssem0).wait()
        pltpu.make_async_copy(
            sbuf.at[1], out_hbm.at[0, :, pl.ds(0, BLK)], ssem1).wait()

    return gather_kernel


def kernel(inputs, table):
    b, t = inputs.shape
    idxt = inputs.T.astype(jnp.int32)
    tablep = table.reshape(-1, TROW)
    outp = _make_gather(b, t)(idxt, tablep)
    return jnp.transpose(outp, (2, 0, 1))
